# baseline (speedup 1.0000x reference)
import functools

import jax
import jax.numpy as jnp
from jax import lax
from jax.experimental import pallas as pl
from jax.experimental.pallas import tpu as pltpu
from jax.experimental.pallas import tpu_sc as plsc

RADIUS = 2
TOP_K = 500
TEMPERATURE = 0.1
KS = 2 * RADIUS + 1
H = W = 512
IMG_PIX = H * W
B = 8
NKP = 512
NW = 32
KPW = (B * NKP) // NW
NG = KPW // 16
PATCH = KS * KS


def _nms_body(x_ref, o_ref):
    x = x_ref[0]
    rm = x
    for s in (1, 2):
        rm = jnp.maximum(rm, jnp.maximum(pltpu.roll(x, s, axis=1),
                                         pltpu.roll(x, W - s, axis=1)))
    cm = rm
    for s in (1, 2):
        cm = jnp.maximum(cm, jnp.maximum(pltpu.roll(rm, s, axis=0),
                                         pltpu.roll(rm, H - s, axis=0)))
    nms = jnp.where(x == cm, x, 0.0)
    rows = jax.lax.broadcasted_iota(jnp.int32, (H, W), 0)
    cols = jax.lax.broadcasted_iota(jnp.int32, (H, W), 1)
    interior = ((rows >= RADIUS) & (rows < H - RADIUS)
                & (cols >= RADIUS) & (cols < W - RADIUS))
    o_ref[0] = jnp.where(interior, nms, 0.0)


CAP = 4096
PXW = (H * W) // 4
NGRP = PXW // 16


@functools.partial(
    pl.kernel,
    mesh=plsc.VectorSubcoreMesh(core_axis_name="c", subcore_axis_name="s"),
    out_type=[jax.ShapeDtypeStruct((NW * CAP,), jnp.float32),
              jax.ShapeDtypeStruct((NW * CAP,), jnp.int32)],
    compiler_params=pltpu.CompilerParams(needs_layout_passes=False),
    scratch_types=[
        pltpu.VMEM((PXW,), jnp.float32),
        pltpu.VMEM((CAP,), jnp.float32),
        pltpu.VMEM((CAP,), jnp.int32),
    ],
)
def _sc_compact(nms_hbm, cval_hbm, cidx_hbm, inbuf, outv, outi):
    wid = lax.axis_index("s") * 2 + lax.axis_index("c")
    pltpu.sync_copy(nms_hbm.at[pl.ds(wid * PXW, PXW)], inbuf)
    strip_base = (wid % (NKP // KPW)) * PXW

    def _zero(i, carry):
        outi[pl.ds(i * 16, 16)] = strip_base + jnp.zeros((16,), jnp.int32)
        return carry
    lax.fori_loop(0, CAP // 16, _zero, jnp.int32(0))

    NCH = 8
    QCAP = CAP // NCH
    QGRP = NGRP // NCH

    def _body(i, cnts):
        new = []
        for q in range(NCH):
            g = q * QGRP + i
            v = inbuf[pl.ds(g * 16, 16)]
            m = v != 0.0
            mi = jnp.where(m, jnp.ones((16,), jnp.int32),
                           jnp.zeros((16,), jnp.int32))
            inc = jnp.sum(mi)
            off = q * QCAP + jnp.minimum(cnts[q], QCAP - 16)
            idxvec = strip_base + g * 16 + lax.iota(jnp.int32, 16)
            plsc.store_compressed(outi.at[pl.ds(off, 16)], idxvec, mask=m)
            new.append(cnts[q] + inc)
        return tuple(new)
    lax.fori_loop(0, QGRP, _body, (jnp.int32(0),) * NCH)

    def _gather(i, carry):
        iv = outi[pl.ds(i * 16, 16)] - strip_base
        outv[pl.ds(i * 16, 16)] = plsc.load_gather(inbuf, [iv])
        return carry
    lax.fori_loop(0, CAP // 16, _gather, jnp.int32(0))

    pltpu.sync_copy(outv, cval_hbm.at[pl.ds(wid * CAP, CAP)])
    pltpu.sync_copy(outi, cidx_hbm.at[pl.ds(wid * CAP, CAP)])


@functools.partial(
    pl.kernel,
    mesh=plsc.VectorSubcoreMesh(core_axis_name="c", subcore_axis_name="s"),
    out_type=[jax.ShapeDtypeStruct((B * NKP,), jnp.float32)] * 4,
    scratch_types=[
        pltpu.VMEM((KPW,), jnp.int32),
        pltpu.VMEM((PATCH, KPW), jnp.int32),
        pltpu.VMEM((PATCH, KPW), jnp.float32),
        pltpu.VMEM((4, KPW), jnp.int32),
        pltpu.VMEM((4, KPW), jnp.float32),
        pltpu.VMEM((4, KPW), jnp.float32),
        pltpu.VMEM((KPW,), jnp.float32),
        pltpu.VMEM((KPW,), jnp.float32),
        pltpu.VMEM((KPW,), jnp.float32),
        pltpu.VMEM((KPW,), jnp.float32),
        pltpu.SemaphoreType.DMA,
    ],
)
def _sc_refine(img_hbm, idx_hbm, okx, oky, odp, osc,
               idx_v, gidx, vals, cidx, cw, cvals, bkx, bky, bdp, bsc, sem):
    wid = lax.axis_index("s") * 2 + lax.axis_index("c")
    base = wid * KPW
    img_base = (wid // (NKP // KPW)) * IMG_PIX

    pltpu.sync_copy(idx_hbm.at[pl.ds(base, KPW)], idx_v)

    for g in range(NG):
        iv = idx_v[pl.ds(g * 16, 16)]
        pos = img_base + iv - (RADIUS * W + RADIUS)
        for j in range(PATCH):
            gidx[j, pl.ds(g * 16, 16)] = pos + ((j // KS) * W + j % KS)

    copies = [pltpu.async_copy(img_hbm.at[gidx.at[j]], vals.at[j], sem)
              for j in range(PATCH)]
    for c in copies:
        c.wait()

    for g in range(NG):
        sl = pl.ds(g * 16, 16)
        vs = [vals[j, sl] for j in range(PATCH)]
        m = vs[0]
        for j in range(1, PATCH):
            m = jnp.maximum(m, vs[j])
        inv_t = 1.0 / TEMPERATURE
        es = [jnp.exp((v - m) * inv_t) for v in vs]
        s = es[0]
        wx = es[0] * float(0 % KS - RADIUS)
        wy = es[0] * float(0 // KS - RADIUS)
        q = es[0] * float((0 % KS - RADIUS) ** 2 + (0 // KS - RADIUS) ** 2)
        for j in range(1, PATCH):
            gx = float(j % KS - RADIUS)
            gy = float(j // KS - RADIUS)
            s = s + es[j]
            wx = wx + es[j] * gx
            wy = wy + es[j] * gy
            c2 = gx * gx + gy * gy
            if c2:
                q = q + es[j] * c2
        xr = wx / s
        yr = wy / s
        num = 0.25 * (q - 2.0 * (xr * wx + yr * wy) + s * (xr * xr + yr * yr))
        bdp[sl] = num / s

        iv = idx_v[sl]
        xf = jnp.bitwise_and(iv, W - 1).astype(jnp.float32)
        yf = lax.shift_right_logical(iv, 9).astype(jnp.float32)
        kx = (xf + xr) / float(W - 1) * 2.0 - 1.0
        ky = (yf + yr) / float(H - 1) * 2.0 - 1.0
        bkx[sl] = kx
        bky[sl] = ky

        px = (kx + 1.0) / 2.0 * float(W - 1)
        py = (ky + 1.0) / 2.0 * float(H - 1)

        def _floor(p):
            t = p.astype(jnp.int32).astype(jnp.float32)
            return jnp.where(p < t, t - 1.0, t)

        x0 = _floor(px)
        y0 = _floor(py)
        x1 = x0 + 1.0
        y1 = y0 + 1.0
        wx1 = px - x0
        wx0 = 1.0 - wx1
        wy1 = py - y0
        wy0 = 1.0 - wy1
        corners = ((y0, x0, wy0 * wx0), (y0, x1, wy0 * wx1),
                   (y1, x0, wy1 * wx0), (y1, x1, wy1 * wx1))
        for ci, (yy, xx, wc) in enumerate(corners):
            valid = ((xx >= 0.0) & (xx <= float(W - 1))
                     & (yy >= 0.0) & (yy <= float(H - 1)))
            xi = jnp.clip(xx, 0.0, float(W - 1)).astype(jnp.int32)
            yi = jnp.clip(yy, 0.0, float(H - 1)).astype(jnp.int32)
            cidx[ci, sl] = img_base + yi * W + xi
            cw[ci, sl] = jnp.where(valid, wc, 0.0)

    ccopies = [pltpu.async_copy(img_hbm.at[cidx.at[ci]], cvals.at[ci], sem)
               for ci in range(4)]
    for c in ccopies:
        c.wait()

    for g in range(NG):
        sl = pl.ds(g * 16, 16)
        acc = cw[0, sl] * cvals[0, sl]
        for ci in range(1, 4):
            acc = acc + cw[ci, sl] * cvals[ci, sl]
        bsc[sl] = acc

    pltpu.sync_copy(bkx, okx.at[pl.ds(base, KPW)])
    pltpu.sync_copy(bky, oky.at[pl.ds(base, KPW)])
    pltpu.sync_copy(bdp, odp.at[pl.ds(base, KPW)])
    pltpu.sync_copy(bsc, osc.at[pl.ds(base, KPW)])


def kernel(scores_map):
    b, _, h, w = scores_map.shape
    imgs = scores_map[:, 0]
    nms = pl.pallas_call(
        _nms_body,
        grid=(b,),
        in_specs=[pl.BlockSpec((1, h, w), lambda i: (i, 0, 0))],
        out_specs=pl.BlockSpec((1, h, w), lambda i: (i, 0, 0)),
        out_shape=jax.ShapeDtypeStruct((b, h, w), jnp.float32),
    )(imgs)

    cval, cidx = _sc_compact(nms.reshape(-1))
    _, pos = jax.lax.top_k(cval.reshape(b, 4 * CAP), TOP_K)
    idx = jnp.take_along_axis(cidx.reshape(b, 4 * CAP), pos, axis=1)

    safe = RADIUS * W + RADIUS
    idx_pad = jnp.pad(idx, ((0, 0), (0, NKP - TOP_K)),
                      constant_values=safe).reshape(-1)

    okx, oky, odp, osc = _sc_refine(imgs.reshape(-1), idx_pad)

    kx = okx.reshape(b, NKP)[:, :TOP_K]
    ky = oky.reshape(b, NKP)[:, :TOP_K]
    keypoints = jnp.stack([kx, ky], axis=-1)
    kptscores = osc.reshape(b, NKP)[:, :TOP_K]
    scoredispersitys = odp.reshape(b, NKP)[:, :TOP_K]
    return keypoints, kptscores, scoredispersitys


# speedup vs baseline: 1.0245x; 1.0245x over previous
import functools

import jax
import jax.numpy as jnp
from jax import lax
from jax.experimental import pallas as pl
from jax.experimental.pallas import tpu as pltpu
from jax.experimental.pallas import tpu_sc as plsc

RADIUS = 2
TOP_K = 500
TEMPERATURE = 0.1
KS = 2 * RADIUS + 1
H = W = 512
HP = H + 2 * RADIUS
WP = W + 2 * RADIUS
IMG_PIX = HP * WP
B = 8
NKP = 512
NW = 32
KPW = (B * NKP) // NW
NG = KPW // 16
PATCH = KS * KS


def _nms_body(x_ref, o_ref):
    x = x_ref[0]
    rm = x
    for s in (1, 2):
        rm = jnp.maximum(rm, jnp.maximum(pltpu.roll(x, s, axis=1),
                                         pltpu.roll(x, W - s, axis=1)))
    cm = rm
    for s in (1, 2):
        cm = jnp.maximum(cm, jnp.maximum(pltpu.roll(rm, s, axis=0),
                                         pltpu.roll(rm, H - s, axis=0)))
    nms = jnp.where(x == cm, x, 0.0)
    rows = jax.lax.broadcasted_iota(jnp.int32, (H, W), 0)
    cols = jax.lax.broadcasted_iota(jnp.int32, (H, W), 1)
    interior = ((rows >= RADIUS) & (rows < H - RADIUS)
                & (cols >= RADIUS) & (cols < W - RADIUS))
    o_ref[0] = jnp.where(interior, nms, 0.0)


CAP = 4096
PXW = (H * W) // 4
NGRP = PXW // 16


@functools.partial(
    pl.kernel,
    mesh=plsc.VectorSubcoreMesh(core_axis_name="c", subcore_axis_name="s"),
    out_type=[jax.ShapeDtypeStruct((NW * CAP,), jnp.float32),
              jax.ShapeDtypeStruct((NW * CAP,), jnp.int32)],
    compiler_params=pltpu.CompilerParams(needs_layout_passes=False),
    scratch_types=[
        pltpu.VMEM((PXW,), jnp.float32),
        pltpu.VMEM((CAP,), jnp.float32),
        pltpu.VMEM((CAP,), jnp.int32),
    ],
)
def _sc_compact(nms_hbm, cval_hbm, cidx_hbm, inbuf, outv, outi):
    wid = lax.axis_index("s") * 2 + lax.axis_index("c")
    pltpu.sync_copy(nms_hbm.at[pl.ds(wid * PXW, PXW)], inbuf)
    strip_base = (wid % (NKP // KPW)) * PXW

    def _zero(i, carry):
        outi[pl.ds(i * 16, 16)] = strip_base + jnp.zeros((16,), jnp.int32)
        return carry
    lax.fori_loop(0, CAP // 16, _zero, jnp.int32(0))

    QCAP = CAP // 4
    QGRP = NGRP // 4

    def _body(i, cnts):
        new = []
        for q in range(4):
            g = q * QGRP + i
            v = inbuf[pl.ds(g * 16, 16)]
            m = v != 0.0
            mi = jnp.where(m, jnp.ones((16,), jnp.int32),
                           jnp.zeros((16,), jnp.int32))
            inc = jnp.sum(mi)
            off = q * QCAP + jnp.minimum(cnts[q], QCAP - 16)
            idxvec = strip_base + g * 16 + lax.iota(jnp.int32, 16)
            plsc.store_compressed(outi.at[pl.ds(off, 16)], idxvec, mask=m)
            new.append(cnts[q] + inc)
        return tuple(new)
    lax.fori_loop(0, QGRP, _body, (jnp.int32(0),) * 4)

    def _gather(i, carry):
        iv = outi[pl.ds(i * 16, 16)] - strip_base
        outv[pl.ds(i * 16, 16)] = plsc.load_gather(inbuf, [iv])
        return carry
    lax.fori_loop(0, CAP // 16, _gather, jnp.int32(0))

    pltpu.sync_copy(outv, cval_hbm.at[pl.ds(wid * CAP, CAP)])
    pltpu.sync_copy(outi, cidx_hbm.at[pl.ds(wid * CAP, CAP)])


@functools.partial(
    pl.kernel,
    mesh=plsc.VectorSubcoreMesh(core_axis_name="c", subcore_axis_name="s"),
    out_type=[jax.ShapeDtypeStruct((B * NKP,), jnp.float32)] * 4,
    scratch_types=[
        pltpu.VMEM((KPW,), jnp.int32),
        pltpu.VMEM((PATCH, KPW), jnp.int32),
        pltpu.VMEM((PATCH, KPW), jnp.float32),
        pltpu.VMEM((4, KPW), jnp.int32),
        pltpu.VMEM((4, KPW), jnp.float32),
        pltpu.VMEM((4, KPW), jnp.float32),
        pltpu.VMEM((KPW,), jnp.float32),
        pltpu.VMEM((KPW,), jnp.float32),
        pltpu.VMEM((KPW,), jnp.float32),
        pltpu.VMEM((KPW,), jnp.float32),
        pltpu.SemaphoreType.DMA,
    ],
)
def _sc_refine(pad_hbm, idx_hbm, okx, oky, odp, osc,
               idx_v, gidx, vals, cidx, cw, cvals, bkx, bky, bdp, bsc, sem):
    wid = lax.axis_index("s") * 2 + lax.axis_index("c")
    base = wid * KPW
    img_base = (wid // (NKP // KPW)) * IMG_PIX

    pltpu.sync_copy(idx_hbm.at[pl.ds(base, KPW)], idx_v)

    for g in range(NG):
        iv = idx_v[pl.ds(g * 16, 16)]
        y = lax.shift_right_logical(iv, 9)
        x = jnp.bitwise_and(iv, W - 1)
        pos = img_base + y * WP + x
        for j in range(PATCH):
            gidx[j, pl.ds(g * 16, 16)] = pos + ((j // KS) * WP + j % KS)

    copies = [pltpu.async_copy(pad_hbm.at[gidx.at[j]], vals.at[j], sem)
              for j in range(PATCH)]
    for c in copies:
        c.wait()

    for g in range(NG):
        sl = pl.ds(g * 16, 16)
        vs = [vals[j, sl] for j in range(PATCH)]
        m = vs[0]
        for j in range(1, PATCH):
            m = jnp.maximum(m, vs[j])
        inv_t = 1.0 / TEMPERATURE
        es = [jnp.exp((v - m) * inv_t) for v in vs]
        s = es[0]
        wx = es[0] * float(0 % KS - RADIUS)
        wy = es[0] * float(0 // KS - RADIUS)
        q = es[0] * float((0 % KS - RADIUS) ** 2 + (0 // KS - RADIUS) ** 2)
        for j in range(1, PATCH):
            gx = float(j % KS - RADIUS)
            gy = float(j // KS - RADIUS)
            s = s + es[j]
            wx = wx + es[j] * gx
            wy = wy + es[j] * gy
            c2 = gx * gx + gy * gy
            if c2:
                q = q + es[j] * c2
        xr = wx / s
        yr = wy / s
        num = 0.25 * (q - 2.0 * (xr * wx + yr * wy) + s * (xr * xr + yr * yr))
        bdp[sl] = num / s

        iv = idx_v[sl]
        xf = jnp.bitwise_and(iv, W - 1).astype(jnp.float32)
        yf = lax.shift_right_logical(iv, 9).astype(jnp.float32)
        kx = (xf + xr) / float(W - 1) * 2.0 - 1.0
        ky = (yf + yr) / float(H - 1) * 2.0 - 1.0
        bkx[sl] = kx
        bky[sl] = ky

        px = (kx + 1.0) / 2.0 * float(W - 1)
        py = (ky + 1.0) / 2.0 * float(H - 1)

        def _floor(p):
            t = p.astype(jnp.int32).astype(jnp.float32)
            return jnp.where(p < t, t - 1.0, t)

        x0 = _floor(px)
        y0 = _floor(py)
        x1 = x0 + 1.0
        y1 = y0 + 1.0
        wx1 = px - x0
        wx0 = 1.0 - wx1
        wy1 = py - y0
        wy0 = 1.0 - wy1
        corners = ((y0, x0, wy0 * wx0), (y0, x1, wy0 * wx1),
                   (y1, x0, wy1 * wx0), (y1, x1, wy1 * wx1))
        for ci, (yy, xx, wc) in enumerate(corners):
            valid = ((xx >= 0.0) & (xx <= float(W - 1))
                     & (yy >= 0.0) & (yy <= float(H - 1)))
            xi = jnp.clip(xx, 0.0, float(W - 1)).astype(jnp.int32)
            yi = jnp.clip(yy, 0.0, float(H - 1)).astype(jnp.int32)
            cidx[ci, sl] = img_base + (yi + RADIUS) * WP + (xi + RADIUS)
            cw[ci, sl] = jnp.where(valid, wc, 0.0)

    ccopies = [pltpu.async_copy(pad_hbm.at[cidx.at[ci]], cvals.at[ci], sem)
               for ci in range(4)]
    for c in ccopies:
        c.wait()

    for g in range(NG):
        sl = pl.ds(g * 16, 16)
        acc = cw[0, sl] * cvals[0, sl]
        for ci in range(1, 4):
            acc = acc + cw[ci, sl] * cvals[ci, sl]
        bsc[sl] = acc

    pltpu.sync_copy(bkx, okx.at[pl.ds(base, KPW)])
    pltpu.sync_copy(bky, oky.at[pl.ds(base, KPW)])
    pltpu.sync_copy(bdp, odp.at[pl.ds(base, KPW)])
    pltpu.sync_copy(bsc, osc.at[pl.ds(base, KPW)])


def kernel(scores_map):
    b, _, h, w = scores_map.shape
    imgs = scores_map[:, 0]
    nms = pl.pallas_call(
        _nms_body,
        grid=(b,),
        in_specs=[pl.BlockSpec((1, h, w), lambda i: (i, 0, 0))],
        out_specs=pl.BlockSpec((1, h, w), lambda i: (i, 0, 0)),
        out_shape=jax.ShapeDtypeStruct((b, h, w), jnp.float32),
    )(imgs)

    cval, cidx = _sc_compact(nms.reshape(-1))
    _, pos = jax.lax.top_k(cval.reshape(b, 4 * CAP), TOP_K)
    idx = jnp.take_along_axis(cidx.reshape(b, 4 * CAP), pos, axis=1)

    r = RADIUS
    padded = jnp.pad(imgs, ((0, 0), (r, r), (r, r)))
    pad_flat = padded.reshape(-1)
    idx_pad = jnp.pad(idx, ((0, 0), (0, NKP - TOP_K))).reshape(-1)

    okx, oky, odp, osc = _sc_refine(pad_flat, idx_pad)

    kx = okx.reshape(b, NKP)[:, :TOP_K]
    ky = oky.reshape(b, NKP)[:, :TOP_K]
    keypoints = jnp.stack([kx, ky], axis=-1)
    kptscores = osc.reshape(b, NKP)[:, :TOP_K]
    scoredispersitys = odp.reshape(b, NKP)[:, :TOP_K]
    return keypoints, kptscores, scoredispersitys
